# baseline (speedup 1.0000x reference)
import jax
import jax.numpy as jnp
from jax import lax
from jax.experimental import pallas as pl
from jax.experimental.pallas import tpu as pltpu
from jax.experimental.pallas import tpu_sc as plsc

D = 64
SCALE = 8.0
NC = 2
NS = 16
NW = NC * NS
B_TOTAL = 16384 * 50
PER_W = B_TOTAL // NW
CHUNK = 512
NCHUNK = PER_W // CHUNK
LANES = 16


def _scale_rows(rows, n):
    def row_body(r, _):
        for j in range(D // LANES):
            sl = pl.ds(j * LANES, LANES)
            rows[r, sl] = rows[r, sl] * SCALE
        return 0
    lax.fori_loop(0, n, row_body, 0, unroll=False)


def _body(x_hbm, lut_hbm, out_hbm, idx_all, rows0, rows1, g0, g1):
    wid = lax.axis_index("s") * NC + lax.axis_index("c")
    base = wid * PER_W

    pltpu.sync_copy(x_hbm.at[pl.ds(base, PER_W)], idx_all)

    rows = (rows0, rows1)
    sems = (g0, g1)

    def start_gather(ci, b):
        pltpu.make_async_copy(
            lut_hbm.at[idx_all.at[pl.ds(ci * CHUNK, CHUNK)]],
            rows[b], sems[b]).start()

    def wait_gather(b):
        pltpu.make_async_copy(
            lut_hbm.at[idx_all.at[pl.ds(0, CHUNK)]],
            rows[b], sems[b]).wait()

    def process(ci, b):
        wait_gather(b)
        _scale_rows(rows[b], CHUNK)
        pltpu.sync_copy(rows[b], out_hbm.at[pl.ds(base + ci * CHUNK, CHUNK)])

    start_gather(0, 0)
    start_gather(1, 1)

    def pair_body(j, _):
        for b in range(2):
            ci = 2 * j + b
            process(ci, b)
            start_gather(ci + 2, b)
        return 0
    lax.fori_loop(0, NCHUNK // 2 - 1, pair_body, 0)

    process(NCHUNK - 2, 0)
    process(NCHUNK - 1, 1)


@jax.jit
def _embed(x_flat, lut):
    mesh = plsc.VectorSubcoreMesh(core_axis_name="c", subcore_axis_name="s")
    return pl.kernel(
        _body,
        out_type=jax.ShapeDtypeStruct((B_TOTAL, D), jnp.float32),
        mesh=mesh,
        compiler_params=pltpu.CompilerParams(use_tc_tiling_on_sc=False),
        scratch_types=[
            pltpu.VMEM((PER_W,), jnp.int32),
            pltpu.VMEM((CHUNK, D), jnp.float32),
            pltpu.VMEM((CHUNK, D), jnp.float32),
            pltpu.SemaphoreType.DMA,
            pltpu.SemaphoreType.DMA,
        ],
    )(x_flat, lut)


def kernel(x, lut):
    x_flat = x.reshape(-1).astype(jnp.int32)
    out = _embed(x_flat, lut)
    return out.reshape(x.shape + (D,))
